# Initial kernel scaffold; baseline (speedup 1.0000x reference)
import jax
import jax.numpy as jnp
from jax.experimental import pallas as pl


def kernel(x, edge_index, bridge_index, W_in, double_attn, bias_param, W_res, prelu_a):
    raise NotImplementedError("write your pallas kernel here")


# speedup vs baseline: 12.7611x; 12.7611x over previous
import functools

import jax
import jax.numpy as jnp
from jax import lax
from jax.experimental import pallas as pl
from jax.experimental.pallas import tpu as pltpu
from jax.experimental.pallas import tpu_sc as plsc

_N = 10000
_NF = 128
_C = 4
_H = 4
_D = 16
_DM = 64
_E = 160000
_B = 10000
_M = _E + 2 * _B
_MP = 180224
_CHB = _MP // 32
_CHC = _MP // 16
_K = 128
_RB = _CHB // _K
_RC = _CHC // _K
_NA = 10240
_NPT = _NA // 16
_NDR = _NA // 8
_NPR = _NA // 2

_f32 = jnp.float32
_i32 = jnp.int32


def _pe_const():
    pos = jnp.arange(_C, dtype=_f32)[:, None]
    di = jnp.arange(0, _DM, 2, dtype=_f32)[None, :]
    div = 10000.0 ** (di / _DM)
    pe = jnp.zeros((_C, _DM), dtype=_f32)
    pe = pe.at[:, 0::2].set(jnp.sin(pos / div))
    pe = pe.at[:, 1::2].set(jnp.cos(pos / div))
    return pe.reshape(1, _C * _DM)


_BN = 400


def _prep_body(x_ref, wt_ref, pe_ref, cat_ref, pair_ref):
    xl = jnp.dot(x_ref[...], wt_ref[...], preferred_element_type=_f32)
    xf = jnp.concatenate([xl, xl, xl, xl], axis=1) + pe_ref[...]
    xt = jnp.swapaxes(xf.reshape(_BN, 16, 16), 1, 2)
    cat_ref[0] = xt.reshape(_BN, 256)
    nxt = jnp.concatenate([xt[:, :, 4:16], xt[:, :, 12:16]], axis=2)
    cat_ref[1] = nxt.reshape(_BN, 256)
    prv = jnp.concatenate([xt[:, :, 0:4], xt[:, :, 0:4], xt[:, :, 4:12]], axis=2)
    cat_ref[2] = prv.reshape(_BN, 256)
    pair_ref[0] = xt[:, 0:8, :].reshape(_BN, 128)
    pair_ref[1] = xt[:, 8:16, :].reshape(_BN, 128)


def _prep_call(x, w_in_t, pe):
    cat3, pair2 = pl.pallas_call(
        _prep_body,
        grid=(_N // _BN,),
        in_specs=[
            pl.BlockSpec((_BN, 128), lambda i: (i, 0)),
            pl.BlockSpec((128, 64), lambda i: (0, 0)),
            pl.BlockSpec((1, 256), lambda i: (0, 0)),
        ],
        out_specs=[
            pl.BlockSpec((3, _BN, 256), lambda i: (0, i, 0)),
            pl.BlockSpec((2, _BN, 128), lambda i: (0, i, 0)),
        ],
        out_shape=[
            jax.ShapeDtypeStruct((3, _N, 256), _f32),
            jax.ShapeDtypeStruct((2, _N, 128), _f32),
        ],
    )(x, w_in_t, pe)
    return cat3.reshape(3 * _N, 256), pair2.reshape(2 * _N, 128)


def _edges_body(xfcat, aidx, bidx, att, edst,
                wexp_out, denomp_out,
                aidx_v, bidx_v, edst_v, idxp_v, urows, vrows, wexp_v, att_v,
                pbuf, dacc, sem):
    cid = lax.axis_index("c")
    sid = lax.axis_index("s")
    wid = cid * 16 + sid
    lane = lax.iota(_i32, 16)

    def _zrow(i, _):
        for j in range(8):
            pbuf[i, pl.ds(j * 16, 16)] = jnp.zeros((16,), _f32)
        return _
    lax.fori_loop(0, _K, _zrow, None)
    nper = _NDR // 16

    def _idxo(i2, _):
        v = sid * nper + i2 * 16 + lane
        idxp_v[pl.ds(i2 * 16, 16)] = jnp.minimum(v, sid * nper + nper - 1)
        return _
    lax.fori_loop(0, _K // 16, _idxo, None)
    pltpu.sync_copy(pbuf, dacc.at[idxp_v])
    plsc.subcore_barrier()

    pltpu.sync_copy(att, att_v)
    atc = [att_v[d, :] for d in range(16)]
    ge12 = jnp.where(lane >= 12, 1.0, 0.0).astype(_f32)
    lt4 = jnp.where(lane < 4, 1.0, 0.0).astype(_f32)
    ones = jnp.ones((16,), _f32)

    def _round(r, _):
        base = wid * _CHB + r * _K
        pltpu.sync_copy(aidx.at[pl.ds(base, _K)], aidx_v)
        pltpu.sync_copy(bidx.at[pl.ds(base, _K)], bidx_v)
        pltpu.sync_copy(edst.at[pl.ds(base, _K)], edst_v)
        pltpu.async_copy(xfcat.at[aidx_v], urows, sem).wait()
        pltpu.async_copy(xfcat.at[bidx_v], vrows, sem).wait()

        def _idxp(i2, _):
            sl = pl.ds(i2 * 16, 16)
            idxp_v[sl] = lax.shift_right_logical(edst_v[sl], 3)
            return _
        lax.fori_loop(0, _K // 16, _idxp, None)

        def _grp(g, _):
            kb = g * 16
            slot16 = lax.rem(edst_v[pl.ds(kb, 16)], 8) * 16
            for kk in range(16):
                k = kb + kk
                p = base + k
                ea = jnp.zeros((16,), _f32)
                for d in range(16):
                    sl = pl.ds(d * 16, 16)
                    s = urows[k, sl] + vrows[k, sl]
                    f = jnp.where(s >= 0.0, s, 0.2 * s)
                    ea = ea + f * atc[d]
                fwd = jnp.where(p >= _E, 1.0, 0.0) * jnp.where(p < _E + _B, 1.0, 0.0)
                bwd = jnp.where(p >= _E + _B, 1.0, 0.0) * jnp.where(p < _M, 1.0, 0.0)
                keep = ones - fwd * ge12 - bwd * lt4
                pad = jnp.where(p >= _M, 0.0, 1.0)
                w = jnp.exp(ea * keep) * pad
                wexp_v[k, :] = w
                for j in range(8):
                    pbuf[k, pl.ds(j * 16, 16)] = jnp.zeros((16,), _f32)
                pbuf[k, pl.ds(slot16[kk], 16)] = w
            return _
        lax.fori_loop(0, _K // 16, _grp, None)

        pltpu.sync_copy(wexp_v, wexp_out.at[pl.ds(base, _K)])
        pltpu.sync_copy(pbuf, dacc.at[idxp_v], add=True)
        return _
    lax.fori_loop(0, _RB, _round, None)

    plsc.subcore_barrier()
    def _idxo2(i2, _):
        v = sid * nper + i2 * 16 + lane
        idxp_v[pl.ds(i2 * 16, 16)] = jnp.minimum(v, sid * nper + nper - 1)
        return _
    lax.fori_loop(0, _K // 16, _idxo2, None)
    pltpu.async_copy(dacc.at[idxp_v], pbuf, sem).wait()
    pltpu.sync_copy(pbuf.at[pl.ds(0, nper)],
                    denomp_out.at[cid, pl.ds(sid * nper, nper)])


def _edges_call(xfcat, aidx, bidx, att, edst):
    mesh = plsc.VectorSubcoreMesh(core_axis_name="c", subcore_axis_name="s")
    f = functools.partial(
        pl.kernel,
        out_type=[
            jax.ShapeDtypeStruct((_MP, 16), _f32),
            jax.ShapeDtypeStruct((2, _NDR, 128), _f32),
        ],
        mesh=mesh,
        scratch_types=[
            pltpu.VMEM((_K,), _i32),
            pltpu.VMEM((_K,), _i32),
            pltpu.VMEM((_K,), _i32),
            pltpu.VMEM((_K,), _i32),
            pltpu.VMEM((_K, 256), _f32),
            pltpu.VMEM((_K, 256), _f32),
            pltpu.VMEM((_K, 16), _f32),
            pltpu.VMEM((16, 16), _f32),
            pltpu.VMEM((_K, 128), _f32),
            pltpu.VMEM_SHARED((_NDR, 128), _f32),
            pltpu.SemaphoreType.DMA,
        ],
    )(_edges_body)
    return f(xfcat, aidx, bidx, att, edst)


_NQ = 2560


def _aggr_body(qh, xfpair, gfull, edst, wexp, denom0, denom1,
               outpair,
               edst_v, g_v, idxp_v, rows_v, wexp_v, d0_v, d1_v, pbuf,
               oacc, sem):
    cid = lax.axis_index("c")
    sid = lax.axis_index("s")
    lane = lax.iota(_i32, 16)
    nper = _NQ // 16
    lo = qh * _NQ

    def _zrow(i, _):
        for j in range(8):
            pbuf[i, pl.ds(j * 16, 16)] = jnp.zeros((16,), _f32)
        return _
    lax.fori_loop(0, _K, _zrow, None)
    for j in range(2):
        def _idxo(i2, _):
            v = sid * nper + j * _K + i2 * 16 + lane
            idxp_v[pl.ds(i2 * 16, 16)] = jnp.minimum(v, sid * nper + nper - 1)
            return _
        lax.fori_loop(0, _K // 16, _idxo, None)
        pltpu.sync_copy(pbuf, oacc.at[idxp_v])
    plsc.subcore_barrier()

    def _round(r, _):
        base = sid * _CHC + r * _K
        pltpu.sync_copy(edst.at[pl.ds(base, _K)], edst_v)
        pltpu.sync_copy(gfull.at[pl.ds(base, _K)], g_v)

        def _off(i, _):
            sl = pl.ds(i * 16, 16)
            g_v[sl] = g_v[sl] + cid * _N
            rel = edst_v[sl] - lo
            idxp_v[sl] = jnp.minimum(jnp.maximum(rel, 0), _NQ - 1)
            return _
        lax.fori_loop(0, _K // 16, _off, None)

        pltpu.async_copy(xfpair.at[g_v], rows_v, sem).wait()
        pltpu.sync_copy(wexp.at[pl.ds(base, _K)], wexp_v)

        def _grp(g, _):
            kb = g * 16
            ev = edst_v[pl.ds(kb, 16)]
            inr16 = (jnp.where(ev >= lo, 1.0, 0.0)
                     * jnp.where(ev < lo + _NQ, 1.0, 0.0)).astype(_f32)
            for kk in range(16):
                k = kb + kk
                wv = wexp_v[k, :] * inr16[kk]
                for j in range(8):
                    sl = pl.ds(j * 16, 16)
                    pbuf[k, sl] = rows_v[k, sl] * wv
            return _
        lax.fori_loop(0, _K // 16, _grp, None)

        pltpu.sync_copy(pbuf, oacc.at[idxp_v], add=True)
        return _
    lax.fori_loop(0, _RC, _round, None)

    plsc.subcore_barrier()
    for j, nrows in ((0, _K), (1, 32)):
        off = sid * nper + j * _K
        def _idxo2(i2, _):
            v = off + i2 * 16 + lane
            idxp_v[pl.ds(i2 * 16, 16)] = jnp.minimum(v, sid * nper + nper - 1)
            return _
        lax.fori_loop(0, _K // 16, _idxo2, None)
        pltpu.async_copy(oacc.at[idxp_v], pbuf, sem).wait()
        pltpu.sync_copy(denom0.at[pl.ds(lo + off, _K)], d0_v)
        pltpu.sync_copy(denom1.at[pl.ds(lo + off, _K)], d1_v)

        def _norm(rr, _):
            winv = 1.0 / (d0_v[rr, :] + d1_v[rr, :] + 1e-16)
            for j2 in range(8):
                sl = pl.ds(j2 * 16, 16)
                pbuf[rr, sl] = pbuf[rr, sl] * winv
            return _
        lax.fori_loop(0, nrows, _norm, None)
        pltpu.sync_copy(pbuf.at[pl.ds(0, nrows)],
                        outpair.at[cid, pl.ds(off, nrows)])
    plsc.subcore_barrier()


def _aggr_call(qh, xfpair, gfull, edst, wexp, denom0, denom1):
    mesh = plsc.VectorSubcoreMesh(core_axis_name="c", subcore_axis_name="s")
    f = functools.partial(
        pl.kernel,
        out_type=jax.ShapeDtypeStruct((2, _NQ, 128), _f32),
        mesh=mesh,
        scratch_types=[
            pltpu.VMEM((_K,), _i32),
            pltpu.VMEM((_K,), _i32),
            pltpu.VMEM((_K,), _i32),
            pltpu.VMEM((_K, 128), _f32),
            pltpu.VMEM((_K, 16), _f32),
            pltpu.VMEM((_K, 16), _f32),
            pltpu.VMEM((_K, 16), _f32),
            pltpu.VMEM((_K, 128), _f32),
            pltpu.VMEM_SHARED((_NQ, 128), _f32),
            pltpu.SemaphoreType.DMA,
        ],
    )(functools.partial(_aggr_body, qh))
    return f(xfpair, gfull, edst, wexp, denom0, denom1)


def _final_body(x_ref, wt_ref, lo_ref, hi_ref, b_ref, a_ref, o_ref):
    res = jnp.dot(x_ref[...], wt_ref[...], preferred_element_type=_f32)
    dstk = jnp.concatenate([lo_ref[...].reshape(_BN, 8, 16),
                            hi_ref[...].reshape(_BN, 8, 16)], axis=1)
    agg = jnp.swapaxes(dstk, 1, 2).reshape(_BN, 256)
    out = agg + res + b_ref[...]
    o_ref[...] = jnp.where(out >= 0.0, out, a_ref[...] * out)


def _final_call(x, w_res_t, lo, hi, bias, a):
    hspec = pl.BlockSpec((_BN, 128), lambda i: (i, 0))
    return pl.pallas_call(
        _final_body,
        grid=(_N // _BN,),
        in_specs=[
            pl.BlockSpec((_BN, 128), lambda i: (i, 0)),
            pl.BlockSpec((128, 256), lambda i: (0, 0)),
            hspec, hspec,
            pl.BlockSpec((1, 256), lambda i: (0, 0)),
            pl.BlockSpec((1, 1), lambda i: (0, 0)),
        ],
        out_specs=pl.BlockSpec((_BN, 256), lambda i: (i, 0)),
        out_shape=jax.ShapeDtypeStruct((_N, 256), _f32),
    )(x, w_res_t, lo, hi, bias, a)


def kernel(x, edge_index, bridge_index, W_in, double_attn, bias_param, W_res, prelu_a):
    pe = _pe_const()
    xfcat, xfpair = _prep_call(x, W_in.T, pe)

    sb, db = edge_index[0], edge_index[1]
    sB, dB = bridge_index[0], bridge_index[1]
    z = jnp.zeros((_MP - _M,), _i32)
    a_idx = jnp.concatenate([sb, dB, sB, z])
    b_idx = jnp.concatenate([db, sB + _N, dB + 2 * _N, z])
    esrc = jnp.concatenate([sb, sB, dB, z])
    edst = jnp.concatenate([db, dB, sB, z])
    att = double_attn[0].T

    gfull = sb[esrc]
    wexp, denomp = _edges_call(xfcat, a_idx, b_idx, att, edst)
    denomf = denomp.reshape(2, _NA, 16)
    parts = [_aggr_call(q, xfpair, gfull, edst, wexp, denomf[0], denomf[1])
             for q in range(4)]
    lo = jnp.concatenate([p[0] for p in parts], axis=0)
    hi = jnp.concatenate([p[1] for p in parts], axis=0)
    out = _final_call(x, W_res.T, lo, hi,
                      bias_param.reshape(1, 256), jnp.reshape(prelu_a, (1, 1)))
    return out


# speedup vs baseline: 12.8928x; 1.0103x over previous
import functools

import jax
import jax.numpy as jnp
from jax import lax
from jax.experimental import pallas as pl
from jax.experimental.pallas import tpu as pltpu
from jax.experimental.pallas import tpu_sc as plsc

_N = 10000
_NF = 128
_C = 4
_H = 4
_D = 16
_DM = 64
_E = 160000
_B = 10000
_M = _E + 2 * _B
_MP = 180224
_CHB = _MP // 32
_CHC = _MP // 16
_K = 128
_RB = _CHB // _K
_RC = _CHC // _K
_NA = 10240
_NPT = _NA // 16
_NDR = _NA // 8
_NPR = _NA // 2

_f32 = jnp.float32
_i32 = jnp.int32


def _pe_const():
    pos = jnp.arange(_C, dtype=_f32)[:, None]
    di = jnp.arange(0, _DM, 2, dtype=_f32)[None, :]
    div = 10000.0 ** (di / _DM)
    pe = jnp.zeros((_C, _DM), dtype=_f32)
    pe = pe.at[:, 0::2].set(jnp.sin(pos / div))
    pe = pe.at[:, 1::2].set(jnp.cos(pos / div))
    return pe.reshape(1, _C * _DM)


_BN = 400


def _prep_body(x_ref, wt_ref, pe_ref, cat_ref, pair_ref):
    xl = jnp.dot(x_ref[...], wt_ref[...], preferred_element_type=_f32)
    xf = jnp.concatenate([xl, xl, xl, xl], axis=1) + pe_ref[...]
    xt = jnp.swapaxes(xf.reshape(_BN, 16, 16), 1, 2)
    cat_ref[0] = xt.reshape(_BN, 256)
    nxt = jnp.concatenate([xt[:, :, 4:16], xt[:, :, 12:16]], axis=2)
    cat_ref[1] = nxt.reshape(_BN, 256)
    prv = jnp.concatenate([xt[:, :, 0:4], xt[:, :, 0:4], xt[:, :, 4:12]], axis=2)
    cat_ref[2] = prv.reshape(_BN, 256)
    pair_ref[0] = xt[:, 0:8, :].reshape(_BN, 128)
    pair_ref[1] = xt[:, 8:16, :].reshape(_BN, 128)


def _prep_call(x, w_in_t, pe):
    cat3, pair2 = pl.pallas_call(
        _prep_body,
        grid=(_N // _BN,),
        in_specs=[
            pl.BlockSpec((_BN, 128), lambda i: (i, 0)),
            pl.BlockSpec((128, 64), lambda i: (0, 0)),
            pl.BlockSpec((1, 256), lambda i: (0, 0)),
        ],
        out_specs=[
            pl.BlockSpec((3, _BN, 256), lambda i: (0, i, 0)),
            pl.BlockSpec((2, _BN, 128), lambda i: (0, i, 0)),
        ],
        out_shape=[
            jax.ShapeDtypeStruct((3, _N, 256), _f32),
            jax.ShapeDtypeStruct((2, _N, 128), _f32),
        ],
    )(x, w_in_t, pe)
    return cat3.reshape(3 * _N, 256), pair2.reshape(2 * _N, 128)


def _edges_body(xfcat, aidx, bidx, att, edst,
                wexp_out, denomp_out,
                aidx_v, bidx_v, edst_v, idxp_v, urows, vrows, wexp_v, att_v,
                pbuf, dacc, sem):
    cid = lax.axis_index("c")
    sid = lax.axis_index("s")
    wid = cid * 16 + sid
    lane = lax.iota(_i32, 16)

    def _zrow(i, _):
        for j in range(8):
            pbuf[i, pl.ds(j * 16, 16)] = jnp.zeros((16,), _f32)
        return _
    lax.fori_loop(0, _K, _zrow, None)
    nper = _NDR // 16

    def _idxo(i2, _):
        v = sid * nper + i2 * 16 + lane
        idxp_v[pl.ds(i2 * 16, 16)] = jnp.minimum(v, sid * nper + nper - 1)
        return _
    lax.fori_loop(0, _K // 16, _idxo, None)
    pltpu.sync_copy(pbuf, dacc.at[idxp_v])
    plsc.subcore_barrier()

    pltpu.sync_copy(att, att_v)
    atc = [att_v[d, :] for d in range(16)]
    ge12 = jnp.where(lane >= 12, 1.0, 0.0).astype(_f32)
    lt4 = jnp.where(lane < 4, 1.0, 0.0).astype(_f32)
    ones = jnp.ones((16,), _f32)

    def _round(r, _):
        base = wid * _CHB + r * _K
        pltpu.sync_copy(aidx.at[pl.ds(base, _K)], aidx_v)
        pltpu.sync_copy(bidx.at[pl.ds(base, _K)], bidx_v)
        pltpu.sync_copy(edst.at[pl.ds(base, _K)], edst_v)
        cu = pltpu.async_copy(xfcat.at[aidx_v], urows, sem)
        cv = pltpu.async_copy(xfcat.at[bidx_v], vrows, sem)
        cu.wait()
        cv.wait()

        def _idxp(i2, _):
            sl = pl.ds(i2 * 16, 16)
            idxp_v[sl] = lax.shift_right_logical(edst_v[sl], 3)
            return _
        lax.fori_loop(0, _K // 16, _idxp, None)

        def _grp(g, _):
            kb = g * 16
            slot16 = lax.rem(edst_v[pl.ds(kb, 16)], 8) * 16
            for kk in range(16):
                k = kb + kk
                p = base + k
                ea = jnp.zeros((16,), _f32)
                for d in range(16):
                    sl = pl.ds(d * 16, 16)
                    s = urows[k, sl] + vrows[k, sl]
                    f = jnp.where(s >= 0.0, s, 0.2 * s)
                    ea = ea + f * atc[d]
                fwd = jnp.where(p >= _E, 1.0, 0.0) * jnp.where(p < _E + _B, 1.0, 0.0)
                bwd = jnp.where(p >= _E + _B, 1.0, 0.0) * jnp.where(p < _M, 1.0, 0.0)
                keep = ones - fwd * ge12 - bwd * lt4
                pad = jnp.where(p >= _M, 0.0, 1.0)
                w = jnp.exp(ea * keep) * pad
                wexp_v[k, :] = w
                for j in range(8):
                    pbuf[k, pl.ds(j * 16, 16)] = jnp.zeros((16,), _f32)
                pbuf[k, pl.ds(slot16[kk], 16)] = w
            return _
        lax.fori_loop(0, _K // 16, _grp, None)

        pltpu.sync_copy(wexp_v, wexp_out.at[pl.ds(base, _K)])
        pltpu.sync_copy(pbuf, dacc.at[idxp_v], add=True)
        return _
    lax.fori_loop(0, _RB, _round, None)

    plsc.subcore_barrier()
    def _idxo2(i2, _):
        v = sid * nper + i2 * 16 + lane
        idxp_v[pl.ds(i2 * 16, 16)] = jnp.minimum(v, sid * nper + nper - 1)
        return _
    lax.fori_loop(0, _K // 16, _idxo2, None)
    pltpu.async_copy(dacc.at[idxp_v], pbuf, sem).wait()
    pltpu.sync_copy(pbuf.at[pl.ds(0, nper)],
                    denomp_out.at[cid, pl.ds(sid * nper, nper)])


def _edges_call(xfcat, aidx, bidx, att, edst):
    mesh = plsc.VectorSubcoreMesh(core_axis_name="c", subcore_axis_name="s")
    f = functools.partial(
        pl.kernel,
        out_type=[
            jax.ShapeDtypeStruct((_MP, 16), _f32),
            jax.ShapeDtypeStruct((2, _NDR, 128), _f32),
        ],
        mesh=mesh,
        scratch_types=[
            pltpu.VMEM((_K,), _i32),
            pltpu.VMEM((_K,), _i32),
            pltpu.VMEM((_K,), _i32),
            pltpu.VMEM((_K,), _i32),
            pltpu.VMEM((_K, 256), _f32),
            pltpu.VMEM((_K, 256), _f32),
            pltpu.VMEM((_K, 16), _f32),
            pltpu.VMEM((16, 16), _f32),
            pltpu.VMEM((_K, 128), _f32),
            pltpu.VMEM_SHARED((_NDR, 128), _f32),
            pltpu.SemaphoreType.DMA,
        ],
    )(_edges_body)
    return f(xfcat, aidx, bidx, att, edst)


_NQ = 2560


def _aggr_body(qh, xfpair, gfull, edst, wexp, denom0, denom1,
               outpair,
               edst_v, g_v, idxp_v, rows_v, wexp_v, d0_v, d1_v, pbuf,
               oacc, sem):
    cid = lax.axis_index("c")
    sid = lax.axis_index("s")
    lane = lax.iota(_i32, 16)
    nper = _NQ // 16
    lo = qh * _NQ

    def _zrow(i, _):
        for j in range(8):
            pbuf[i, pl.ds(j * 16, 16)] = jnp.zeros((16,), _f32)
        return _
    lax.fori_loop(0, _K, _zrow, None)
    for j in range(2):
        def _idxo(i2, _):
            v = sid * nper + j * _K + i2 * 16 + lane
            idxp_v[pl.ds(i2 * 16, 16)] = jnp.minimum(v, sid * nper + nper - 1)
            return _
        lax.fori_loop(0, _K // 16, _idxo, None)
        pltpu.sync_copy(pbuf, oacc.at[idxp_v])
    plsc.subcore_barrier()

    def _round(r, _):
        base = sid * _CHC + r * _K
        pltpu.sync_copy(edst.at[pl.ds(base, _K)], edst_v)
        pltpu.sync_copy(gfull.at[pl.ds(base, _K)], g_v)

        def _off(i, _):
            sl = pl.ds(i * 16, 16)
            g_v[sl] = g_v[sl] + cid * _N
            rel = edst_v[sl] - lo
            idxp_v[sl] = jnp.minimum(jnp.maximum(rel, 0), _NQ - 1)
            return _
        lax.fori_loop(0, _K // 16, _off, None)

        pltpu.async_copy(xfpair.at[g_v], rows_v, sem).wait()
        pltpu.sync_copy(wexp.at[pl.ds(base, _K)], wexp_v)

        def _grp(g, _):
            kb = g * 16
            ev = edst_v[pl.ds(kb, 16)]
            inr16 = (jnp.where(ev >= lo, 1.0, 0.0)
                     * jnp.where(ev < lo + _NQ, 1.0, 0.0)).astype(_f32)
            for kk in range(16):
                k = kb + kk
                wv = wexp_v[k, :] * inr16[kk]
                for j in range(8):
                    sl = pl.ds(j * 16, 16)
                    pbuf[k, sl] = rows_v[k, sl] * wv
            return _
        lax.fori_loop(0, _K // 16, _grp, None)

        pltpu.sync_copy(pbuf, oacc.at[idxp_v], add=True)
        return _
    lax.fori_loop(0, _RC, _round, None)

    plsc.subcore_barrier()
    for j, nrows in ((0, _K), (1, 32)):
        off = sid * nper + j * _K
        def _idxo2(i2, _):
            v = off + i2 * 16 + lane
            idxp_v[pl.ds(i2 * 16, 16)] = jnp.minimum(v, sid * nper + nper - 1)
            return _
        lax.fori_loop(0, _K // 16, _idxo2, None)
        pltpu.async_copy(oacc.at[idxp_v], pbuf, sem).wait()
        pltpu.sync_copy(denom0.at[pl.ds(lo + off, _K)], d0_v)
        pltpu.sync_copy(denom1.at[pl.ds(lo + off, _K)], d1_v)

        def _norm(rr, _):
            winv = 1.0 / (d0_v[rr, :] + d1_v[rr, :] + 1e-16)
            for j2 in range(8):
                sl = pl.ds(j2 * 16, 16)
                pbuf[rr, sl] = pbuf[rr, sl] * winv
            return _
        lax.fori_loop(0, nrows, _norm, None)
        pltpu.sync_copy(pbuf.at[pl.ds(0, nrows)],
                        outpair.at[cid, pl.ds(off, nrows)])
    plsc.subcore_barrier()


def _aggr_call(qh, xfpair, gfull, edst, wexp, denom0, denom1):
    mesh = plsc.VectorSubcoreMesh(core_axis_name="c", subcore_axis_name="s")
    f = functools.partial(
        pl.kernel,
        out_type=jax.ShapeDtypeStruct((2, _NQ, 128), _f32),
        mesh=mesh,
        scratch_types=[
            pltpu.VMEM((_K,), _i32),
            pltpu.VMEM((_K,), _i32),
            pltpu.VMEM((_K,), _i32),
            pltpu.VMEM((_K, 128), _f32),
            pltpu.VMEM((_K, 16), _f32),
            pltpu.VMEM((_K, 16), _f32),
            pltpu.VMEM((_K, 16), _f32),
            pltpu.VMEM((_K, 128), _f32),
            pltpu.VMEM_SHARED((_NQ, 128), _f32),
            pltpu.SemaphoreType.DMA,
        ],
    )(functools.partial(_aggr_body, qh))
    return f(xfpair, gfull, edst, wexp, denom0, denom1)


def _final_body(x_ref, wt_ref, lo_ref, hi_ref, b_ref, a_ref, o_ref):
    res = jnp.dot(x_ref[...], wt_ref[...], preferred_element_type=_f32)
    dstk = jnp.concatenate([lo_ref[...].reshape(_BN, 8, 16),
                            hi_ref[...].reshape(_BN, 8, 16)], axis=1)
    agg = jnp.swapaxes(dstk, 1, 2).reshape(_BN, 256)
    out = agg + res + b_ref[...]
    o_ref[...] = jnp.where(out >= 0.0, out, a_ref[...] * out)


def _final_call(x, w_res_t, lo, hi, bias, a):
    hspec = pl.BlockSpec((_BN, 128), lambda i: (i, 0))
    return pl.pallas_call(
        _final_body,
        grid=(_N // _BN,),
        in_specs=[
            pl.BlockSpec((_BN, 128), lambda i: (i, 0)),
            pl.BlockSpec((128, 256), lambda i: (0, 0)),
            hspec, hspec,
            pl.BlockSpec((1, 256), lambda i: (0, 0)),
            pl.BlockSpec((1, 1), lambda i: (0, 0)),
        ],
        out_specs=pl.BlockSpec((_BN, 256), lambda i: (i, 0)),
        out_shape=jax.ShapeDtypeStruct((_N, 256), _f32),
    )(x, w_res_t, lo, hi, bias, a)


def kernel(x, edge_index, bridge_index, W_in, double_attn, bias_param, W_res, prelu_a):
    pe = _pe_const()
    xfcat, xfpair = _prep_call(x, W_in.T, pe)

    sb, db = edge_index[0], edge_index[1]
    sB, dB = bridge_index[0], bridge_index[1]
    z = jnp.zeros((_MP - _M,), _i32)
    a_idx = jnp.concatenate([sb, dB, sB, z])
    b_idx = jnp.concatenate([db, sB + _N, dB + 2 * _N, z])
    esrc = jnp.concatenate([sb, sB, dB, z])
    edst = jnp.concatenate([db, dB, sB, z])
    att = double_attn[0].T

    gfull = sb[esrc]
    wexp, denomp = _edges_call(xfcat, a_idx, b_idx, att, edst)
    denomf = denomp.reshape(2, _NA, 16)
    parts = [_aggr_call(q, xfpair, gfull, edst, wexp, denomf[0], denomf[1])
             for q in range(4)]
    lo = jnp.concatenate([p[0] for p in parts], axis=0)
    hi = jnp.concatenate([p[1] for p in parts], axis=0)
    out = _final_call(x, W_res.T, lo, hi,
                      bias_param.reshape(1, 256), jnp.reshape(prelu_a, (1, 1)))
    return out
